# baseline (speedup 1.0000x reference)
import functools

import jax
import jax.numpy as jnp
import numpy as np
from jax import lax
from jax.experimental import pallas as pl
from jax.experimental.pallas import tpu as pltpu
from jax.experimental.pallas import tpu_sc as plsc

NC = 2
NS = 16
NW = NC * NS
LANES = 16

B = 80
SB = 25
DEPTH = 3
WRITERS = 10
WR = 1000
BL = 2000
GRID = 5


def _deg_body(dst_hbm, ew_hbm, out_hbm, acc_v, dst_v, ew_v, ew_per):
    c = lax.axis_index("c")
    s = lax.axis_index("s")
    wid = c * NS + s
    n = acc_v.shape[0]

    def zero(i, _):
        acc_v[pl.ds(i * LANES, LANES)] = jnp.zeros((LANES,), jnp.float32)
        return _

    lax.fori_loop(0, n // LANES, zero, None)

    pltpu.sync_copy(dst_hbm.at[pl.ds(wid * ew_per, ew_per)], dst_v)
    pltpu.sync_copy(ew_hbm.at[pl.ds(wid * ew_per, ew_per)], ew_v)

    def group(k, _):
        sl = pl.ds(k * LANES, LANES)
        plsc.addupdate_scatter(acc_v, [dst_v[sl]], ew_v[sl])
        return _

    lax.fori_loop(0, ew_per // LANES, group, None)
    for g in range(GRID):
        pltpu.sync_copy(acc_v.at[pl.ds(g * BL, BL)],
                        out_hbm.at[pl.ds((g * NW + wid) * BL, BL)])


def _agg_body(y_hbm, src_hbm, dst_hbm, ew_hbm, z_hbm, out_hbm,
              acc_sh, src_v, dst_v, ew_v, srcq,
              rows0, rows1, rows2, msg0, msg1,
              gsem0, gsem1, gsem2, msem0, msem1, nsb, n):
    c = lax.axis_index("c")
    s = lax.axis_index("s")
    wid = c * NS + s
    bufs = (rows0, rows1, rows2)
    gsems = (gsem0, gsem1, gsem2)
    msgs = (msg0, msg1)
    msems = (msem0, msem1)
    h = msg0.shape[1]
    nbatch = nsb * SB
    sbw = SB * B

    @pl.when(s < WRITERS)
    def _zero_fill():
        pltpu.sync_copy(z_hbm, acc_sh.at[pl.ds(s * WR, WR)])

    plsc.subcore_barrier()

    def wait_g(t):
        pltpu.make_async_copy(y_hbm.at[srcq.at[pl.ds(0, B)]],
                              bufs[t], gsems[t]).wait()

    def wait_s(i):
        pltpu.make_async_copy(msgs[i], acc_sh.at[dst_v.at[0]], msems[i]).wait()

    def issue_gather(t, q):
        qr = lax.rem(q, SB)
        for g in range(B // LANES):
            srcq[pl.ds(t * B + g * LANES, LANES)] = (
                src_v[pl.ds(qr * B + g * LANES, LANES)])
        pltpu.async_copy(y_hbm.at[srcq.at[pl.ds(t * B, B)]], bufs[t], gsems[t])

    pltpu.sync_copy(src_hbm.at[pl.ds(wid * nbatch * B, sbw)], src_v)
    pltpu.sync_copy(dst_hbm.at[wid, 0], dst_v)
    pltpu.sync_copy(ew_hbm.at[pl.ds(wid * nbatch * B, sbw)], ew_v)
    for t in range(2):
        issue_gather(t, t)

    def batch(bi, _):
        k = lax.rem(bi, DEPTH)
        sbi = lax.div(bi, SB)
        r = lax.rem(bi, SB)
        nxt = bi + 2
        m2 = lax.rem(nxt, DEPTH)

        for t in range(DEPTH):
            @pl.when(jnp.logical_and(m2 == t, nxt < nbatch))
            def _refill(t=t):
                issue_gather(t, nxt)

        @pl.when(jnp.logical_and(r == SB - 3, sbi + 1 < nsb))
        def _stage_src():
            pltpu.sync_copy(
                src_hbm.at[pl.ds((wid * nsb + sbi + 1) * sbw, sbw)], src_v)

        for t in range(DEPTH):
            for mi in range(2):
                @pl.when(jnp.logical_and(k == t, lax.rem(bi, 2) == mi))
                def _go(t=t, mi=mi):
                    @pl.when(jnp.logical_and(bi >= 2, r != 1))
                    def _drain():
                        wait_s(mi)

                    wait_g(t)
                    wbase = r * B
                    buf = bufs[t]
                    msg = msgs[mi]

                    @plsc.parallel_loop(0, B, unroll=2)
                    def _scale(j):
                        w16 = plsc.load_gather(
                            ew_v, [jnp.full((LANES,), wbase + j, jnp.int32)])
                        for f in range(h // 32):
                            vi = buf[j, pl.ds(f * LANES, LANES)]
                            v32 = plsc.bitcast(vi, jnp.bfloat16)
                            va, vb = plsc.unpack(
                                v32, format=plsc.PackFormat.INTERLEAVED)
                            msg[j, pl.ds(f * 32, LANES)] = va * w16
                            msg[j, pl.ds(f * 32 + LANES, LANES)] = vb * w16

                    pltpu.async_copy(msg, acc_sh.at[dst_v.at[r]], msems[mi],
                                     add=True)

                    @pl.when(r == SB - 1)
                    def _boundary():
                        wait_s(mi)

                        @pl.when(sbi + 1 < nsb)
                        def _stage_dst_ew():
                            pltpu.sync_copy(dst_hbm.at[wid, sbi + 1], dst_v)
                            pltpu.sync_copy(
                                ew_hbm.at[
                                    pl.ds((wid * nsb + sbi + 1) * sbw, sbw)],
                                ew_v)

        return _

    lax.fori_loop(0, nbatch, batch, None)
    wait_s((nbatch - 2) % 2)
    plsc.subcore_barrier()

    @pl.when(s < WRITERS)
    def _writeout():
        sl = pl.ds(s * WR, WR)
        pltpu.sync_copy(acc_sh.at[sl], out_hbm.at[c, sl])


def _block_dis(degp_ref):
    deg = jnp.sum(degp_ref[0], axis=0) + 1.0
    return lax.rsqrt(deg)


def _prep_body(x_ref, w_ref, wp_ref, degp_ref, y_ref, ybf_ref):
    dis = _block_dis(degp_ref)
    xw = jnp.dot(x_ref[...], w_ref[...], preferred_element_type=jnp.float32)
    y_ref[...] = xw * dis[:, None]
    xwp = jnp.dot(x_ref[...], wp_ref[...], preferred_element_type=jnp.float32)
    ybf_ref[...] = (xwp * dis[:, None]).astype(jnp.bfloat16)


def _final_body(aggp_ref, y_ref, degp_ref, b_ref, emb_ref, relu_ref):
    dis = _block_dis(degp_ref)
    agg = aggp_ref[0] + aggp_ref[1] + y_ref[...]
    emb = agg * dis[:, None] + b_ref[...]
    emb_ref[...] = emb
    relu_ref[...] = jnp.maximum(emb, 0.0)


def kernel(x, level, edge_index, edge_weight, W, b):
    del level
    n, f_in = x.shape
    h = W.shape[1]
    e = edge_weight.shape[0]
    assert e % (NW * B * SB) == 0 and n == WRITERS * WR
    nb = e // (NW * B * SB)

    src_flat = edge_index[0].astype(jnp.int32)
    dst_flat = edge_index[1].astype(jnp.int32)
    dst = dst_flat.reshape(NW, nb, SB, B)
    ew_per = e // NW

    mesh = plsc.VectorSubcoreMesh(core_axis_name="c", subcore_axis_name="s")

    deg_partial = pl.kernel(
        functools.partial(_deg_body, ew_per=ew_per),
        out_type=jax.ShapeDtypeStruct((NW * n,), jnp.float32),
        mesh=mesh,
        scratch_types=[
            pltpu.VMEM((n,), jnp.float32),
            pltpu.VMEM((ew_per,), jnp.int32),
            pltpu.VMEM((ew_per,), jnp.float32),
        ],
        compiler_params=pltpu.CompilerParams(needs_layout_passes=False),
        name="sc_deg_scatter",
    )(dst_flat, edge_weight)
    bl, grid = BL, GRID
    degp_t = deg_partial.reshape(grid, NW, bl)

    pcols = np.empty(h, np.int32)
    for f0 in range(0, h, 32):
        for i in range(16):
            pcols[f0 + 2 * i] = f0 + i
            pcols[f0 + 2 * i + 1] = f0 + 16 + i

    y, ybf = pl.pallas_call(
        _prep_body,
        grid=(grid,),
        in_specs=[
            pl.BlockSpec((bl, f_in), lambda i: (i, 0)),
            pl.BlockSpec((f_in, h), lambda i: (0, 0)),
            pl.BlockSpec((f_in, h), lambda i: (0, 0)),
            pl.BlockSpec((1, NW, bl), lambda i: (i, 0, 0)),
        ],
        out_specs=[
            pl.BlockSpec((bl, h), lambda i: (i, 0)),
            pl.BlockSpec((bl, h), lambda i: (i, 0)),
        ],
        out_shape=[
            jax.ShapeDtypeStruct((n, h), jnp.float32),
            jax.ShapeDtypeStruct((n, h), jnp.bfloat16),
        ],
        name="tc_prep_matmul",
    )(x, W, W[:, pcols], degp_t)
    ybf32 = lax.bitcast_convert_type(ybf.reshape(n, h // 2, 2), jnp.int32)

    agg_partial = pl.kernel(
        functools.partial(_agg_body, nsb=nb, n=n),
        out_type=jax.ShapeDtypeStruct((NC, n, h), jnp.float32),
        mesh=mesh,
        scratch_types=[
            pltpu.VMEM_SHARED((n, h), jnp.float32),
            pltpu.VMEM((SB * B,), jnp.int32),
            pltpu.VMEM((SB, B), jnp.int32),
            pltpu.VMEM((SB * B,), jnp.float32),
            pltpu.VMEM((DEPTH * B,), jnp.int32),
            pltpu.VMEM((B, h // 2), jnp.int32),
            pltpu.VMEM((B, h // 2), jnp.int32),
            pltpu.VMEM((B, h // 2), jnp.int32),
            pltpu.VMEM((B, h), jnp.float32),
            pltpu.VMEM((B, h), jnp.float32),
            pltpu.SemaphoreType.DMA,
            pltpu.SemaphoreType.DMA,
            pltpu.SemaphoreType.DMA,
            pltpu.SemaphoreType.DMA,
            pltpu.SemaphoreType.DMA,
        ],
        compiler_params=pltpu.CompilerParams(needs_layout_passes=False,
                                             use_tc_tiling_on_sc=False),
        name="sc_edge_aggregate",
    )(ybf32, src_flat, dst, edge_weight, jnp.zeros((WR, h), jnp.float32))

    embedding, to_next = pl.pallas_call(
        _final_body,
        grid=(grid,),
        in_specs=[
            pl.BlockSpec((NC, bl, h), lambda i: (0, i, 0)),
            pl.BlockSpec((bl, h), lambda i: (i, 0)),
            pl.BlockSpec((1, NW, bl), lambda i: (i, 0, 0)),
            pl.BlockSpec((1, h), lambda i: (0, 0)),
        ],
        out_specs=[
            pl.BlockSpec((bl, h), lambda i: (i, 0)),
            pl.BlockSpec((bl, h), lambda i: (i, 0)),
        ],
        out_shape=[
            jax.ShapeDtypeStruct((n, h), jnp.float32),
            jax.ShapeDtypeStruct((n, h), jnp.float32),
        ],
        name="tc_finalize",
    )(agg_partial, y, degp_t, b.reshape(1, h))

    return (embedding, to_next)


# speedup vs baseline: 1.0058x; 1.0058x over previous
import functools

import jax
import jax.numpy as jnp
import numpy as np
from jax import lax
from jax.experimental import pallas as pl
from jax.experimental.pallas import tpu as pltpu
from jax.experimental.pallas import tpu_sc as plsc

NC = 2
NS = 16
NW = NC * NS
LANES = 16

B = 80
SB = 25
DEPTH = 3
WRITERS = 10
WR = 1000
BL = 2000
GRID = 5


def _deg_body(dst_hbm, ew_hbm, out_hbm, acc_v, dst_v, ew_v, ew_per):
    c = lax.axis_index("c")
    s = lax.axis_index("s")
    wid = c * NS + s
    n = acc_v.shape[0]

    def zero(i, _):
        acc_v[pl.ds(i * LANES, LANES)] = jnp.zeros((LANES,), jnp.float32)
        return _

    lax.fori_loop(0, n // LANES, zero, None)

    pltpu.sync_copy(dst_hbm.at[pl.ds(wid * ew_per, ew_per)], dst_v)
    pltpu.sync_copy(ew_hbm.at[pl.ds(wid * ew_per, ew_per)], ew_v)

    def group(k, _):
        sl = pl.ds(k * LANES, LANES)
        plsc.addupdate_scatter(acc_v, [dst_v[sl]], ew_v[sl])
        return _

    lax.fori_loop(0, ew_per // LANES, group, None)
    for g in range(GRID):
        pltpu.sync_copy(acc_v.at[pl.ds(g * BL, BL)],
                        out_hbm.at[pl.ds((g * NW + wid) * BL, BL)])


def _agg_body(y_hbm, src_hbm, dst_hbm, ew_hbm, z_hbm, out_hbm,
              acc_sh, src_v, dst_v, ew_v, srcq, dstq,
              rows0, rows1, rows2, msg0, msg1,
              gsem0, gsem1, gsem2, msem0, msem1, nsb, n):
    c = lax.axis_index("c")
    s = lax.axis_index("s")
    wid = c * NS + s
    bufs = (rows0, rows1, rows2)
    gsems = (gsem0, gsem1, gsem2)
    msgs = (msg0, msg1)
    msems = (msem0, msem1)
    h = msg0.shape[1]
    nbatch = nsb * SB
    sbw = SB * B

    @pl.when(s < WRITERS)
    def _zero_fill():
        pltpu.sync_copy(z_hbm, acc_sh.at[pl.ds(s * WR, WR)])

    plsc.subcore_barrier()

    def wait_g(t):
        pltpu.make_async_copy(y_hbm.at[srcq.at[pl.ds(0, B)]],
                              bufs[t], gsems[t]).wait()

    def wait_s(i):
        pltpu.make_async_copy(msgs[i], acc_sh.at[dstq.at[0]], msems[i]).wait()

    def issue_gather(t, q):
        qr = lax.rem(q, SB)
        for g in range(B // LANES):
            srcq[pl.ds(t * B + g * LANES, LANES)] = (
                src_v[pl.ds(qr * B + g * LANES, LANES)])
        pltpu.async_copy(y_hbm.at[srcq.at[pl.ds(t * B, B)]], bufs[t], gsems[t])

    pltpu.sync_copy(src_hbm.at[pl.ds(wid * nbatch * B, sbw)], src_v)
    pltpu.sync_copy(dst_hbm.at[pl.ds(wid * nbatch * B, sbw)], dst_v)
    pltpu.sync_copy(ew_hbm.at[pl.ds(wid * nbatch * B, sbw)], ew_v)
    for t in range(2):
        issue_gather(t, t)

    def batch(bi, _):
        k = lax.rem(bi, DEPTH)
        sbi = lax.div(bi, SB)
        r = lax.rem(bi, SB)
        nxt = bi + 2
        m2 = lax.rem(nxt, DEPTH)

        for t in range(DEPTH):
            @pl.when(jnp.logical_and(m2 == t, nxt < nbatch))
            def _refill(t=t):
                issue_gather(t, nxt)

        @pl.when(jnp.logical_and(r == SB - 3, sbi + 1 < nsb))
        def _stage_src():
            pltpu.sync_copy(
                src_hbm.at[pl.ds((wid * nsb + sbi + 1) * sbw, sbw)], src_v)

        for t in range(DEPTH):
            for mi in range(2):
                @pl.when(jnp.logical_and(k == t, lax.rem(bi, 2) == mi))
                def _go(t=t, mi=mi):
                    @pl.when(bi >= 2)
                    def _drain():
                        wait_s(mi)

                    wait_g(t)
                    wbase = r * B
                    buf = bufs[t]
                    msg = msgs[mi]

                    @plsc.parallel_loop(0, B, unroll=2)
                    def _scale(j):
                        w16 = plsc.load_gather(
                            ew_v, [jnp.full((LANES,), wbase + j, jnp.int32)])
                        for f in range(h // 32):
                            vi = buf[j, pl.ds(f * LANES, LANES)]
                            v32 = plsc.bitcast(vi, jnp.bfloat16)
                            va, vb = plsc.unpack(
                                v32, format=plsc.PackFormat.INTERLEAVED)
                            msg[j, pl.ds(f * 32, LANES)] = va * w16
                            msg[j, pl.ds(f * 32 + LANES, LANES)] = vb * w16

                    for g in range(B // LANES):
                        dstq[mi, pl.ds(g * LANES, LANES)] = (
                            dst_v[pl.ds(r * B + g * LANES, LANES)])

                    pltpu.async_copy(msg, acc_sh.at[dstq.at[mi]], msems[mi],
                                     add=True)

                    @pl.when(jnp.logical_and(r == SB - 1, sbi + 1 < nsb))
                    def _stage_dst_ew():
                        off = pl.ds((wid * nsb + sbi + 1) * sbw, sbw)
                        pltpu.sync_copy(dst_hbm.at[off], dst_v)
                        pltpu.sync_copy(ew_hbm.at[off], ew_v)

        return _

    lax.fori_loop(0, nbatch, batch, None)
    wait_s((nbatch - 2) % 2)
    wait_s((nbatch - 1) % 2)
    plsc.subcore_barrier()

    @pl.when(s < WRITERS)
    def _writeout():
        sl = pl.ds(s * WR, WR)
        pltpu.sync_copy(acc_sh.at[sl], out_hbm.at[c, sl])


def _block_dis(degp_ref):
    deg = jnp.sum(degp_ref[0], axis=0) + 1.0
    return lax.rsqrt(deg)


def _prep_body(x_ref, w_ref, wp_ref, degp_ref, y_ref, ybf_ref):
    dis = _block_dis(degp_ref)
    xw = jnp.dot(x_ref[...], w_ref[...], preferred_element_type=jnp.float32)
    y_ref[...] = xw * dis[:, None]
    xwp = jnp.dot(x_ref[...], wp_ref[...], preferred_element_type=jnp.float32)
    ybf_ref[...] = (xwp * dis[:, None]).astype(jnp.bfloat16)


def _final_body(aggp_ref, y_ref, degp_ref, b_ref, emb_ref, relu_ref):
    dis = _block_dis(degp_ref)
    agg = aggp_ref[0] + aggp_ref[1] + y_ref[...]
    emb = agg * dis[:, None] + b_ref[...]
    emb_ref[...] = emb
    relu_ref[...] = jnp.maximum(emb, 0.0)


def kernel(x, level, edge_index, edge_weight, W, b):
    del level
    n, f_in = x.shape
    h = W.shape[1]
    e = edge_weight.shape[0]
    assert e % (NW * B * SB) == 0 and n == WRITERS * WR
    nb = e // (NW * B * SB)

    src_flat = edge_index[0].astype(jnp.int32)
    dst_flat = edge_index[1].astype(jnp.int32)
    ew_per = e // NW

    mesh = plsc.VectorSubcoreMesh(core_axis_name="c", subcore_axis_name="s")

    deg_partial = pl.kernel(
        functools.partial(_deg_body, ew_per=ew_per),
        out_type=jax.ShapeDtypeStruct((NW * n,), jnp.float32),
        mesh=mesh,
        scratch_types=[
            pltpu.VMEM((n,), jnp.float32),
            pltpu.VMEM((ew_per,), jnp.int32),
            pltpu.VMEM((ew_per,), jnp.float32),
        ],
        compiler_params=pltpu.CompilerParams(needs_layout_passes=False),
        name="sc_deg_scatter",
    )(dst_flat, edge_weight)
    bl, grid = BL, GRID
    degp_t = deg_partial.reshape(grid, NW, bl)

    pcols = np.empty(h, np.int32)
    for f0 in range(0, h, 32):
        for i in range(16):
            pcols[f0 + 2 * i] = f0 + i
            pcols[f0 + 2 * i + 1] = f0 + 16 + i

    y, ybf = pl.pallas_call(
        _prep_body,
        grid=(grid,),
        in_specs=[
            pl.BlockSpec((bl, f_in), lambda i: (i, 0)),
            pl.BlockSpec((f_in, h), lambda i: (0, 0)),
            pl.BlockSpec((f_in, h), lambda i: (0, 0)),
            pl.BlockSpec((1, NW, bl), lambda i: (i, 0, 0)),
        ],
        out_specs=[
            pl.BlockSpec((bl, h), lambda i: (i, 0)),
            pl.BlockSpec((bl, h), lambda i: (i, 0)),
        ],
        out_shape=[
            jax.ShapeDtypeStruct((n, h), jnp.float32),
            jax.ShapeDtypeStruct((n, h), jnp.bfloat16),
        ],
        name="tc_prep_matmul",
    )(x, W, W[:, pcols], degp_t)
    ybf32 = lax.bitcast_convert_type(ybf.reshape(n, h // 2, 2), jnp.int32)

    agg_partial = pl.kernel(
        functools.partial(_agg_body, nsb=nb, n=n),
        out_type=jax.ShapeDtypeStruct((NC, n, h), jnp.float32),
        mesh=mesh,
        scratch_types=[
            pltpu.VMEM_SHARED((n, h), jnp.float32),
            pltpu.VMEM((SB * B,), jnp.int32),
            pltpu.VMEM((SB * B,), jnp.int32),
            pltpu.VMEM((SB * B,), jnp.float32),
            pltpu.VMEM((DEPTH * B,), jnp.int32),
            pltpu.VMEM((2, B), jnp.int32),
            pltpu.VMEM((B, h // 2), jnp.int32),
            pltpu.VMEM((B, h // 2), jnp.int32),
            pltpu.VMEM((B, h // 2), jnp.int32),
            pltpu.VMEM((B, h), jnp.float32),
            pltpu.VMEM((B, h), jnp.float32),
            pltpu.SemaphoreType.DMA,
            pltpu.SemaphoreType.DMA,
            pltpu.SemaphoreType.DMA,
            pltpu.SemaphoreType.DMA,
            pltpu.SemaphoreType.DMA,
        ],
        compiler_params=pltpu.CompilerParams(needs_layout_passes=False,
                                             use_tc_tiling_on_sc=False),
        name="sc_edge_aggregate",
    )(ybf32, src_flat, dst_flat, edge_weight, jnp.zeros((WR, h), jnp.float32))

    embedding, to_next = pl.pallas_call(
        _final_body,
        grid=(grid,),
        in_specs=[
            pl.BlockSpec((NC, bl, h), lambda i: (0, i, 0)),
            pl.BlockSpec((bl, h), lambda i: (i, 0)),
            pl.BlockSpec((1, NW, bl), lambda i: (i, 0, 0)),
            pl.BlockSpec((1, h), lambda i: (0, 0)),
        ],
        out_specs=[
            pl.BlockSpec((bl, h), lambda i: (i, 0)),
            pl.BlockSpec((bl, h), lambda i: (i, 0)),
        ],
        out_shape=[
            jax.ShapeDtypeStruct((n, h), jnp.float32),
            jax.ShapeDtypeStruct((n, h), jnp.float32),
        ],
        name="tc_finalize",
    )(agg_partial, y, degp_t, b.reshape(1, h))

    return (embedding, to_next)
